# Initial kernel scaffold; baseline (speedup 1.0000x reference)
import jax
import jax.numpy as jnp
from jax.experimental import pallas as pl


def kernel(x, table, W, b):
    raise NotImplementedError("write your pallas kernel here")


# speedup vs baseline: 1.0699x; 1.0699x over previous
import functools

import jax
import jax.numpy as jnp
from jax import lax
from jax.experimental import pallas as pl
from jax.experimental.pallas import tpu as pltpu
from jax.experimental.pallas import tpu_sc as plsc

MAX_IDX = 62
DLEV = 8
NDIM = 768
VPAD = 64


def _fuse_body(table_ref, w_ref, b_ref, out_ref):
    t = table_ref[...]
    w = w_ref[...]
    acc = lax.dot_general(t, w, (((1,), (1,)), ((), ())),
                          preferred_element_type=jnp.float32)
    z = acc + b_ref[...][None, :]
    out_ref[...] = 0.5 * z * (1.0 + lax.erf(z * (2.0 ** -0.5)))


def _fused_table(table, w, b):
    tpad = jnp.zeros((VPAD, DLEV), jnp.float32).at[:table.shape[0]].set(table)
    return pl.pallas_call(
        _fuse_body,
        out_shape=jax.ShapeDtypeStruct((VPAD, NDIM), jnp.float32),
    )(tpad, w, b)


def _make_gather(n_rows):
    info = plsc.get_sparse_core_info()
    nw = info.num_cores * info.num_subcores
    chunk = 128
    assert n_rows % (nw * chunk) == 0
    rows_per_w = n_rows // nw
    n_chunks = rows_per_w // chunk
    mesh = plsc.VectorSubcoreMesh(core_axis_name="c", subcore_axis_name="s")

    @functools.partial(
        pl.kernel,
        out_type=jax.ShapeDtypeStruct((n_rows, NDIM), jnp.float32),
        mesh=mesh,
        scratch_types=[
            pltpu.VMEM((chunk,), jnp.int32),
            pltpu.VMEM((chunk, NDIM), jnp.float32),
            pltpu.SemaphoreType.DMA,
        ],
    )
    def gather(fused_hbm, idx_hbm, out_hbm, idx_v, rows_v, sem):
        wid = lax.axis_index("s") * info.num_cores + lax.axis_index("c")
        w_base = wid * rows_per_w

        def body(i, carry):
            base = w_base + i * chunk
            pltpu.sync_copy(idx_hbm.at[pl.ds(base, chunk)], idx_v)
            pltpu.async_copy(fused_hbm.at[idx_v], rows_v, sem).wait()
            pltpu.sync_copy(rows_v, out_hbm.at[pl.ds(base, chunk)])
            return carry

        lax.fori_loop(0, n_chunks, body, 0)

    return gather


def kernel(x, table, W, b):
    bsz, seq = x.shape
    fused = _fused_table(table, W, b)
    idx = x.reshape(-1).astype(jnp.int32)
    out = _make_gather(idx.shape[0])(fused, idx)
    return out.reshape(bsz, seq, NDIM)


# speedup vs baseline: 1.4263x; 1.3331x over previous
import functools

import jax
import jax.numpy as jnp
from jax import lax
from jax.experimental import pallas as pl
from jax.experimental.pallas import tpu as pltpu
from jax.experimental.pallas import tpu_sc as plsc

MAX_IDX = 62
DLEV = 8
NDIM = 768
VPAD = 64


def _fuse_body(table_ref, w_ref, b_ref, out_ref):
    t = table_ref[...]
    w = w_ref[...]
    acc = lax.dot_general(t, w, (((1,), (1,)), ((), ())),
                          preferred_element_type=jnp.float32)
    z = acc + b_ref[...][None, :]
    out_ref[...] = 0.5 * z * (1.0 + lax.erf(z * (2.0 ** -0.5)))


def _fused_table(table, w, b):
    tpad = jnp.zeros((VPAD, DLEV), jnp.float32).at[:table.shape[0]].set(table)
    return pl.pallas_call(
        _fuse_body,
        out_shape=jax.ShapeDtypeStruct((VPAD, NDIM), jnp.float32),
    )(tpad, w, b)


def _make_gather(n_rows):
    info = plsc.get_sparse_core_info()
    nw = info.num_cores * info.num_subcores
    chunk = 128
    assert n_rows % (nw * chunk) == 0
    rows_per_w = n_rows // nw
    n_chunks = rows_per_w // chunk
    mesh = plsc.VectorSubcoreMesh(core_axis_name="c", subcore_axis_name="s")

    @functools.partial(
        pl.kernel,
        out_type=jax.ShapeDtypeStruct((n_rows, NDIM), jnp.float32),
        mesh=mesh,
        scratch_types=[
            pltpu.VMEM((chunk,), jnp.int32),
            pltpu.VMEM((chunk, NDIM), jnp.float32),
            pltpu.SemaphoreType.DMA,
        ],
    )
    def gather(fused_hbm, idx_hbm, out_hbm, idx_v, rows_v, sem):
        wid = lax.axis_index("s") * info.num_cores + lax.axis_index("c")
        w_base = wid * rows_per_w
        row_off = wid * VPAD

        def body(i, carry):
            base = w_base + i * chunk
            pltpu.sync_copy(idx_hbm.at[pl.ds(base, chunk)], idx_v)
            for j in range(chunk // 16):
                sl = pl.ds(j * 16, 16)
                idx_v[sl] = idx_v[sl] + row_off
            pltpu.async_copy(fused_hbm.at[idx_v], rows_v, sem).wait()
            pltpu.sync_copy(rows_v, out_hbm.at[pl.ds(base, chunk)])
            return carry

        lax.fori_loop(0, n_chunks, body, 0)

    return gather


def kernel(x, table, W, b):
    bsz, seq = x.shape
    info = plsc.get_sparse_core_info()
    nw = info.num_cores * info.num_subcores
    fused = _fused_table(table, W, b)
    fused_rep = jnp.tile(fused, (nw, 1))
    idx = x.reshape(-1).astype(jnp.int32)
    out = _make_gather(idx.shape[0])(fused_rep, idx)
    return out.reshape(bsz, seq, NDIM)


# speedup vs baseline: 1.4527x; 1.0185x over previous
import functools

import jax
import jax.numpy as jnp
from jax import lax
from jax.experimental import pallas as pl
from jax.experimental.pallas import tpu as pltpu
from jax.experimental.pallas import tpu_sc as plsc

MAX_IDX = 62
DLEV = 8
NDIM = 768
VPAD = 64


def _fuse_body(table_ref, w_ref, b_ref, out_ref):
    t = table_ref[...]
    w = w_ref[...]
    acc = lax.dot_general(t, w, (((1,), (1,)), ((), ())),
                          preferred_element_type=jnp.float32)
    z = acc + b_ref[...][None, :]
    out_ref[...] = 0.5 * z * (1.0 + lax.erf(z * (2.0 ** -0.5)))


def _fused_table(table, w, b):
    tpad = jnp.zeros((VPAD, DLEV), jnp.float32).at[:table.shape[0]].set(table)
    return pl.pallas_call(
        _fuse_body,
        out_shape=jax.ShapeDtypeStruct((VPAD, NDIM), jnp.float32),
    )(tpad, w, b)


def _make_gather(n_rows):
    info = plsc.get_sparse_core_info()
    nw = info.num_cores * info.num_subcores
    chunk = 64
    assert n_rows % (nw * chunk) == 0
    rows_per_w = n_rows // nw
    n_chunks = rows_per_w // chunk
    mesh = plsc.VectorSubcoreMesh(core_axis_name="c", subcore_axis_name="s")

    @functools.partial(
        pl.kernel,
        out_type=jax.ShapeDtypeStruct((n_rows, NDIM), jnp.float32),
        mesh=mesh,
        scratch_types=[
            pltpu.VMEM((rows_per_w,), jnp.int32),
            pltpu.VMEM((2, chunk, NDIM), jnp.float32),
            pltpu.SemaphoreType.DMA((2,)),
            pltpu.SemaphoreType.DMA((2,)),
        ],
    )
    def gather(fused_hbm, idx_hbm, out_hbm, idx_v, rows2, sem_g, sem_w):
        wid = lax.axis_index("s") * info.num_cores + lax.axis_index("c")
        w_base = wid * rows_per_w
        row_off = wid * VPAD

        pltpu.sync_copy(idx_hbm.at[pl.ds(w_base, rows_per_w)], idx_v)

        def bias(j, carry):
            sl = pl.ds(j * 16, 16)
            idx_v[sl] = idx_v[sl] + row_off
            return carry

        lax.fori_loop(0, rows_per_w // 16, bias, 0)

        def g_desc(i, p):
            return pltpu.make_async_copy(
                fused_hbm.at[idx_v.at[pl.ds(i * chunk, chunk)]],
                rows2.at[p], sem_g.at[p])

        def w_desc(i, p):
            return pltpu.make_async_copy(
                rows2.at[p], out_hbm.at[pl.ds(w_base + i * chunk, chunk)],
                sem_w.at[p])

        g_desc(0, 0).start()

        def body(i, carry):
            p = lax.rem(i, 2)
            q = 1 - p
            g_desc(i, p).wait()

            @pl.when(i >= 1)
            def _():
                w_desc(i - 1, q).wait()

            @pl.when(i + 1 < n_chunks)
            def _():
                g_desc(i + 1, q).start()

            w_desc(i, p).start()
            return carry

        lax.fori_loop(0, n_chunks, body, 0)
        w_desc(n_chunks - 1, lax.rem(n_chunks - 1, 2)).wait()

    return gather


def kernel(x, table, W, b):
    bsz, seq = x.shape
    info = plsc.get_sparse_core_info()
    nw = info.num_cores * info.num_subcores
    fused = _fused_table(table, W, b)
    fused_rep = jnp.tile(fused, (nw, 1))
    idx = x.reshape(-1).astype(jnp.int32)
    out = _make_gather(idx.shape[0])(fused_rep, idx)
    return out.reshape(bsz, seq, NDIM)
